# Initial kernel scaffold; baseline (speedup 1.0000x reference)
import functools

import jax
import jax.numpy as jnp
from jax import lax
from jax.experimental import pallas as pl
from jax.experimental.pallas import tpu as pltpu
from jax.experimental.pallas import tpu_sc as plsc

N = 10000
E = 320000
D = 128

NC = 2
NS = 16
NW = NC * NS
EW = E // NW
CH = 125
NCHUNK = EW // CH
ROWS_W = N // NS
WCH = 125
NWCH = ROWS_W // WCH

_mesh = plsc.VectorSubcoreMesh(core_axis_name="c", subcore_axis_name="s",
                               num_cores=NC)


def _worker(c, s):
    return c * NS + s


@functools.partial(
    pl.kernel,
    out_type=jax.ShapeDtypeStruct((NC, N, 16), jnp.float32),
    mesh=_mesh,
    scratch_types=[
        pltpu.VMEM_SHARED((N, 16), jnp.float32),
        pltpu.VMEM((NCHUNK, CH), jnp.int32),
        pltpu.VMEM((CH, 16), jnp.float32),
        pltpu.VMEM((WCH, 16), jnp.float32),
    ],
)
def _sc_deg(rows_hbm, out_hbm, acc_sh, idx_v, ones_v, buf_v):
    c = lax.axis_index("c")
    s = lax.axis_index("s")
    w = _worker(c, s)

    lane0 = jnp.where(lax.iota(jnp.int32, 16) == 0, 1.0, 0.0)
    zero16 = jnp.zeros((16,), jnp.float32)

    def fill(i, _):
        ones_v[i, :] = lane0
        buf_v[i, :] = zero16
        return 0

    lax.fori_loop(0, CH, fill, 0)

    def zslab(k, _):
        pltpu.sync_copy(buf_v, acc_sh.at[pl.ds(s * ROWS_W + k * WCH, WCH)])
        return 0

    lax.fori_loop(0, NWCH, zslab, 0)
    plsc.subcore_barrier()

    pltpu.sync_copy(rows_hbm.at[w], idx_v)

    def step(j, _):
        pltpu.sync_copy(ones_v, acc_sh.at[idx_v.at[j]], add=True)
        return 0

    lax.fori_loop(0, NCHUNK, step, 0)
    plsc.subcore_barrier()

    def wslab(k, _):
        base = s * ROWS_W + k * WCH
        pltpu.sync_copy(acc_sh.at[pl.ds(base, WCH)], buf_v)
        pltpu.sync_copy(buf_v, out_hbm.at[c, pl.ds(base, WCH)])
        return 0

    lax.fori_loop(0, NWCH, wslab, 0)


@functools.partial(
    pl.kernel,
    out_type=jax.ShapeDtypeStruct((NC, N, D), jnp.float32),
    mesh=_mesh,
    scratch_types=[
        pltpu.VMEM_SHARED((N, D), jnp.float32),
        pltpu.VMEM((NCHUNK, CH), jnp.int32),
        pltpu.VMEM((NCHUNK, CH), jnp.int32),
        pltpu.VMEM((CH, D), jnp.float32),
        pltpu.VMEM((WCH, D), jnp.float32),
        pltpu.SemaphoreType.DMA,
    ],
)
def _sc_prop(g_hbm, rows_hbm, cols_hbm, out_hbm, acc_sh, ridx_v, cidx_v,
             rows_v, buf_v, sem):
    c = lax.axis_index("c")
    s = lax.axis_index("s")
    w = _worker(c, s)

    zero16 = jnp.zeros((16,), jnp.float32)

    def zrow(i, _):
        def zcol(k, _):
            buf_v[i, pl.ds(k * 16, 16)] = zero16
            return 0
        lax.fori_loop(0, D // 16, zcol, 0)
        return 0

    lax.fori_loop(0, WCH, zrow, 0)

    def zslab(k, _):
        pltpu.sync_copy(buf_v, acc_sh.at[pl.ds(s * ROWS_W + k * WCH, WCH)])
        return 0

    lax.fori_loop(0, NWCH, zslab, 0)
    plsc.subcore_barrier()

    pltpu.sync_copy(rows_hbm.at[w], ridx_v)
    pltpu.sync_copy(cols_hbm.at[w], cidx_v)

    def step(j, _):
        pltpu.async_copy(g_hbm.at[ridx_v.at[j]], rows_v, sem).wait()
        pltpu.sync_copy(rows_v, acc_sh.at[cidx_v.at[j]], add=True)
        return 0

    lax.fori_loop(0, NCHUNK, step, 0)
    plsc.subcore_barrier()

    def wslab(k, _):
        base = s * ROWS_W + k * WCH
        pltpu.sync_copy(acc_sh.at[pl.ds(base, WCH)], buf_v)
        pltpu.sync_copy(buf_v, out_hbm.at[c, pl.ds(base, WCH)])
        return 0

    lax.fori_loop(0, NWCH, wslab, 0)


BM = 2000
GRID = (N // BM,)


def _tc_pre(degp, x):
    def body(degp_ref, x_ref, dis_ref, g_ref):
        d = degp_ref[0, :, 0:1] + degp_ref[1, :, 0:1]
        dis = jnp.where(d > 0, lax.rsqrt(jnp.maximum(d, 1.0)), 0.0)
        dis_ref[...] = dis
        g_ref[...] = dis * x_ref[...]

    return pl.pallas_call(
        body,
        grid=GRID,
        in_specs=[pl.BlockSpec((NC, BM, 16), lambda i: (0, i, 0)),
                  pl.BlockSpec((BM, D), lambda i: (i, 0))],
        out_specs=[pl.BlockSpec((BM, 1), lambda i: (i, 0)),
                   pl.BlockSpec((BM, D), lambda i: (i, 0))],
        out_shape=[jax.ShapeDtypeStruct((N, 1), jnp.float32),
                   jax.ShapeDtypeStruct((N, D), jnp.float32)],
    )(degp, x)


def _tc_comb(sp, dis):
    def body(sp_ref, dis_ref, t_ref, g_ref):
        dis = dis_ref[...]
        t = -dis * (sp_ref[0] + sp_ref[1])
        t_ref[...] = t
        g_ref[...] = dis * t

    return pl.pallas_call(
        body,
        grid=GRID,
        in_specs=[pl.BlockSpec((NC, BM, D), lambda i: (0, i, 0)),
                  pl.BlockSpec((BM, 1), lambda i: (i, 0))],
        out_specs=[pl.BlockSpec((BM, D), lambda i: (i, 0)),
                   pl.BlockSpec((BM, D), lambda i: (i, 0))],
        out_shape=[jax.ShapeDtypeStruct((N, D), jnp.float32),
                   jax.ShapeDtypeStruct((N, D), jnp.float32)],
    )(sp, dis)


def _tc_layer(xin, t1, sp2, dis, w0, w1, w2, b, want_relu):
    def body(x_ref, t1_ref, sp2_ref, dis_ref, w0_ref, w1_ref, w2_ref, b_ref,
             *out_refs):
        x = x_ref[...]
        dis = dis_ref[...]
        t2 = -2.0 * dis * (sp2_ref[0] + sp2_ref[1]) - x
        acc = jnp.dot(x, w0_ref[...], preferred_element_type=jnp.float32)
        acc += jnp.dot(t1_ref[...], w1_ref[...],
                       preferred_element_type=jnp.float32)
        acc += jnp.dot(t2, w2_ref[...], preferred_element_type=jnp.float32)
        acc += b_ref[...]
        if want_relu:
            h = jnp.maximum(acc, 0.0)
            out_refs[0][...] = h
            out_refs[1][...] = dis * h
        else:
            out_refs[0][...] = acc

    nout = 2 if want_relu else 1
    full = pl.BlockSpec((D, D), lambda i: (0, 0))
    res = pl.pallas_call(
        body,
        grid=GRID,
        in_specs=[pl.BlockSpec((BM, D), lambda i: (i, 0)),
                  pl.BlockSpec((BM, D), lambda i: (i, 0)),
                  pl.BlockSpec((NC, BM, D), lambda i: (0, i, 0)),
                  pl.BlockSpec((BM, 1), lambda i: (i, 0)),
                  full, full, full,
                  pl.BlockSpec((1, D), lambda i: (0, 0))],
        out_specs=[pl.BlockSpec((BM, D), lambda i: (i, 0))] * nout,
        out_shape=[jax.ShapeDtypeStruct((N, D), jnp.float32)] * nout,
    )(xin, t1, sp2, dis, w0, w1, w2, b)
    return res


def kernel(x, edge_index, W10, W11, W12, b1, W20, W21, W22, b2):
    rows = edge_index[0].astype(jnp.int32).reshape(NW, NCHUNK, CH)
    cols = edge_index[1].astype(jnp.int32).reshape(NW, NCHUNK, CH)
    b1r = b1.reshape(1, D)
    b2r = b2.reshape(1, D)

    degp = _sc_deg(rows)
    dis, g1 = _tc_pre(degp, x)
    s1 = _sc_prop(g1, rows, cols)
    t1, g2 = _tc_comb(s1, dis)
    s2 = _sc_prop(g2, rows, cols)
    h, g3 = _tc_layer(x, t1, s2, dis, W10, W11, W12, b1r, True)
    s3 = _sc_prop(g3, rows, cols)
    u1, g4 = _tc_comb(s3, dis)
    s4 = _sc_prop(g4, rows, cols)
    (out,) = _tc_layer(h, u1, s4, dis, W20, W21, W22, b2r, False)
    return out


# speedup vs baseline: 12.6011x; 12.6011x over previous
import functools

import jax
import jax.numpy as jnp
from jax import lax
from jax.experimental import pallas as pl
from jax.experimental.pallas import tpu as pltpu
from jax.experimental.pallas import tpu_sc as plsc

N = 10000
E = 320000
D = 128

NC = 2
NS = 16
NW = NC * NS
EW = E // NW
CH = 125
NCHUNK = EW // CH
WCH = 200
NWCH = N // WCH
WPT = -(-NWCH // NS)
WCHP = 80
NWCHP = N // WCHP
WPTP = -(-NWCHP // NS)

_mesh = plsc.VectorSubcoreMesh(core_axis_name="c", subcore_axis_name="s",
                               num_cores=NC)


def _worker(c, s):
    return c * NS + s


@functools.partial(
    pl.kernel,
    out_type=jax.ShapeDtypeStruct((NC, N, D), jnp.float32),
    mesh=_mesh,
    scratch_types=[
        pltpu.VMEM_SHARED((N, D), jnp.float32),
        pltpu.VMEM((NCHUNK, CH), jnp.int32),
        pltpu.VMEM((CH, D), jnp.float32),
    ],
)
def _sc_deg(rows_hbm, out_hbm, acc_sh, idx_v, ones_v):
    c = lax.axis_index("c")
    s = lax.axis_index("s")
    w = _worker(c, s)

    one16 = jnp.ones((16,), jnp.float32)
    zero16 = jnp.zeros((16,), jnp.float32)

    def frow(i, _):
        def fcol(k, _):
            ones_v[i, pl.ds(k * 16, 16)] = zero16
            return 0
        lax.fori_loop(0, D // 16, fcol, 0)
        return 0

    lax.fori_loop(0, CH, frow, 0)

    def zslab(k, _):
        cid = k * NS + s

        @pl.when(cid < NWCHP)
        def _():
            pltpu.sync_copy(ones_v.at[pl.ds(0, WCHP)],
                            acc_sh.at[pl.ds(cid * WCHP, WCHP)])
        return 0

    lax.fori_loop(0, WPTP, zslab, 0)
    plsc.subcore_barrier()

    def orow(i, _):
        def ocol(k, _):
            ones_v[i, pl.ds(k * 16, 16)] = one16
            return 0
        lax.fori_loop(0, D // 16, ocol, 0)
        return 0

    lax.fori_loop(0, CH, orow, 0)

    pltpu.sync_copy(rows_hbm.at[w], idx_v)

    def step(j, _):
        pltpu.sync_copy(ones_v, acc_sh.at[idx_v.at[j]], add=True)
        return 0

    lax.fori_loop(0, NCHUNK, step, 0)
    plsc.subcore_barrier()

    def wslab(k, _):
        cid = k * NS + s

        @pl.when(cid < NWCHP)
        def _():
            base = cid * WCHP
            pltpu.sync_copy(acc_sh.at[pl.ds(base, WCHP)],
                            ones_v.at[pl.ds(0, WCHP)])
            pltpu.sync_copy(ones_v.at[pl.ds(0, WCHP)],
                            out_hbm.at[c, pl.ds(base, WCHP)])
        return 0

    lax.fori_loop(0, WPTP, wslab, 0)


@functools.partial(
    pl.kernel,
    out_type=jax.ShapeDtypeStruct((NC, N, D), jnp.float32),
    mesh=_mesh,
    scratch_types=[
        pltpu.VMEM_SHARED((N, D), jnp.float32),
        pltpu.VMEM((NCHUNK, CH), jnp.int32),
        pltpu.VMEM((NCHUNK, CH), jnp.int32),
        pltpu.VMEM((CH, D), jnp.float32),
        pltpu.SemaphoreType.DMA,
    ],
)
def _sc_prop(g_hbm, rows_hbm, cols_hbm, out_hbm, acc_sh, ridx_v, cidx_v,
             rows_v, sem):
    c = lax.axis_index("c")
    s = lax.axis_index("s")
    w = _worker(c, s)

    zero16 = jnp.zeros((16,), jnp.float32)

    def zrow(i, _):
        def zcol(k, _):
            rows_v[i, pl.ds(k * 16, 16)] = zero16
            return 0
        lax.fori_loop(0, D // 16, zcol, 0)
        return 0

    lax.fori_loop(0, CH, zrow, 0)

    def zslab(k, _):
        cid = k * NS + s

        @pl.when(cid < NWCHP)
        def _():
            pltpu.sync_copy(rows_v.at[pl.ds(0, WCHP)],
                            acc_sh.at[pl.ds(cid * WCHP, WCHP)])
        return 0

    lax.fori_loop(0, WPTP, zslab, 0)
    plsc.subcore_barrier()

    pltpu.sync_copy(rows_hbm.at[w], ridx_v)
    pltpu.sync_copy(cols_hbm.at[w], cidx_v)

    def step(j, _):
        pltpu.async_copy(g_hbm.at[ridx_v.at[j]], rows_v, sem).wait()
        pltpu.sync_copy(rows_v, acc_sh.at[cidx_v.at[j]], add=True)
        return 0

    lax.fori_loop(0, NCHUNK, step, 0)
    plsc.subcore_barrier()

    def wslab(k, _):
        cid = k * NS + s

        @pl.when(cid < NWCHP)
        def _():
            base = cid * WCHP
            pltpu.sync_copy(acc_sh.at[pl.ds(base, WCHP)],
                            rows_v.at[pl.ds(0, WCHP)])
            pltpu.sync_copy(rows_v.at[pl.ds(0, WCHP)],
                            out_hbm.at[c, pl.ds(base, WCHP)])
        return 0

    lax.fori_loop(0, WPTP, wslab, 0)


BM = 2000
GRID = (N // BM,)


def _tc_pre(degp, x):
    def body(degp_ref, x_ref, dis_ref, g_ref):
        d = degp_ref[0, :, 0:1] + degp_ref[1, :, 0:1]
        dis = jnp.where(d > 0, lax.rsqrt(jnp.maximum(d, 1.0)), 0.0)
        dis_ref[...] = dis
        g_ref[...] = dis * x_ref[...]

    return pl.pallas_call(
        body,
        grid=GRID,
        in_specs=[pl.BlockSpec((NC, BM, D), lambda i: (0, i, 0)),
                  pl.BlockSpec((BM, D), lambda i: (i, 0))],
        out_specs=[pl.BlockSpec((BM, 1), lambda i: (i, 0)),
                   pl.BlockSpec((BM, D), lambda i: (i, 0))],
        out_shape=[jax.ShapeDtypeStruct((N, 1), jnp.float32),
                   jax.ShapeDtypeStruct((N, D), jnp.float32)],
    )(degp, x)


def _tc_comb(sp, dis):
    def body(sp_ref, dis_ref, t_ref, g_ref):
        dis = dis_ref[...]
        t = -dis * (sp_ref[0] + sp_ref[1])
        t_ref[...] = t
        g_ref[...] = dis * t

    return pl.pallas_call(
        body,
        grid=GRID,
        in_specs=[pl.BlockSpec((NC, BM, D), lambda i: (0, i, 0)),
                  pl.BlockSpec((BM, 1), lambda i: (i, 0))],
        out_specs=[pl.BlockSpec((BM, D), lambda i: (i, 0)),
                   pl.BlockSpec((BM, D), lambda i: (i, 0))],
        out_shape=[jax.ShapeDtypeStruct((N, D), jnp.float32),
                   jax.ShapeDtypeStruct((N, D), jnp.float32)],
    )(sp, dis)


def _tc_layer(xin, t1, sp2, dis, w0, w1, w2, b, want_relu):
    def body(x_ref, t1_ref, sp2_ref, dis_ref, w0_ref, w1_ref, w2_ref, b_ref,
             *out_refs):
        x = x_ref[...]
        dis = dis_ref[...]
        t2 = -2.0 * dis * (sp2_ref[0] + sp2_ref[1]) - x
        acc = jnp.dot(x, w0_ref[...], preferred_element_type=jnp.float32)
        acc += jnp.dot(t1_ref[...], w1_ref[...],
                       preferred_element_type=jnp.float32)
        acc += jnp.dot(t2, w2_ref[...], preferred_element_type=jnp.float32)
        acc += b_ref[...]
        if want_relu:
            h = jnp.maximum(acc, 0.0)
            out_refs[0][...] = h
            out_refs[1][...] = dis * h
        else:
            out_refs[0][...] = acc

    nout = 2 if want_relu else 1
    full = pl.BlockSpec((D, D), lambda i: (0, 0))
    res = pl.pallas_call(
        body,
        grid=GRID,
        in_specs=[pl.BlockSpec((BM, D), lambda i: (i, 0)),
                  pl.BlockSpec((BM, D), lambda i: (i, 0)),
                  pl.BlockSpec((NC, BM, D), lambda i: (0, i, 0)),
                  pl.BlockSpec((BM, 1), lambda i: (i, 0)),
                  full, full, full,
                  pl.BlockSpec((1, D), lambda i: (0, 0))],
        out_specs=[pl.BlockSpec((BM, D), lambda i: (i, 0))] * nout,
        out_shape=[jax.ShapeDtypeStruct((N, D), jnp.float32)] * nout,
    )(xin, t1, sp2, dis, w0, w1, w2, b)
    return res


def kernel(x, edge_index, W10, W11, W12, b1, W20, W21, W22, b2):
    rows = edge_index[0].astype(jnp.int32).reshape(NW, NCHUNK, CH)
    cols = edge_index[1].astype(jnp.int32).reshape(NW, NCHUNK, CH)
    b1r = b1.reshape(1, D)
    b2r = b2.reshape(1, D)

    degp = _sc_deg(rows)
    dis, g1 = _tc_pre(degp, x)
    s1 = _sc_prop(g1, rows, cols)
    t1, g2 = _tc_comb(s1, dis)
    s2 = _sc_prop(g2, rows, cols)
    h, g3 = _tc_layer(x, t1, s2, dis, W10, W11, W12, b1r, True)
    s3 = _sc_prop(g3, rows, cols)
    u1, g4 = _tc_comb(s3, dis)
    s4 = _sc_prop(g4, rows, cols)
    (out,) = _tc_layer(h, u1, s4, dis, W20, W21, W22, b2r, False)
    return out


# speedup vs baseline: 15.0926x; 1.1977x over previous
import functools

import jax
import jax.numpy as jnp
from jax import lax
from jax.experimental import pallas as pl
from jax.experimental.pallas import tpu as pltpu
from jax.experimental.pallas import tpu_sc as plsc

N = 10000
E = 320000
D = 128

NC = 2
NS = 16
NW = NC * NS
EW = E // NW
CH = 100
NCHUNK = EW // CH
WCH = 200
NWCH = N // WCH
WPT = -(-NWCH // NS)
WCHP = 80
NWCHP = N // WCHP
WPTP = -(-NWCHP // NS)

_mesh = plsc.VectorSubcoreMesh(core_axis_name="c", subcore_axis_name="s",
                               num_cores=NC)


def _worker(c, s):
    return c * NS + s


@functools.partial(
    pl.kernel,
    out_type=jax.ShapeDtypeStruct((NC, N, D), jnp.float32),
    mesh=_mesh,
    scratch_types=[
        pltpu.VMEM_SHARED((N, D), jnp.float32),
        pltpu.VMEM((NCHUNK, CH), jnp.int32),
        pltpu.VMEM((CH, D), jnp.float32),
    ],
)
def _sc_deg(rows_hbm, out_hbm, acc_sh, idx_v, ones_v):
    c = lax.axis_index("c")
    s = lax.axis_index("s")
    w = _worker(c, s)

    one16 = jnp.ones((16,), jnp.float32)
    zero16 = jnp.zeros((16,), jnp.float32)

    def frow(i, _):
        def fcol(k, _):
            ones_v[i, pl.ds(k * 16, 16)] = zero16
            return 0
        lax.fori_loop(0, D // 16, fcol, 0)
        return 0

    lax.fori_loop(0, CH, frow, 0)

    def zslab(k, _):
        cid = k * NS + s

        @pl.when(cid < NWCHP)
        def _():
            pltpu.sync_copy(ones_v.at[pl.ds(0, WCHP)],
                            acc_sh.at[pl.ds(cid * WCHP, WCHP)])
        return 0

    lax.fori_loop(0, WPTP, zslab, 0)
    plsc.subcore_barrier()

    def orow(i, _):
        def ocol(k, _):
            ones_v[i, pl.ds(k * 16, 16)] = one16
            return 0
        lax.fori_loop(0, D // 16, ocol, 0)
        return 0

    lax.fori_loop(0, CH, orow, 0)

    pltpu.sync_copy(rows_hbm.at[w], idx_v)

    def step(j, _):
        pltpu.sync_copy(ones_v, acc_sh.at[idx_v.at[j]], add=True)
        return 0

    lax.fori_loop(0, NCHUNK, step, 0)
    plsc.subcore_barrier()

    def wslab(k, _):
        cid = k * NS + s

        @pl.when(cid < NWCHP)
        def _():
            base = cid * WCHP
            pltpu.sync_copy(acc_sh.at[pl.ds(base, WCHP)],
                            ones_v.at[pl.ds(0, WCHP)])
            pltpu.sync_copy(ones_v.at[pl.ds(0, WCHP)],
                            out_hbm.at[c, pl.ds(base, WCHP)])
        return 0

    lax.fori_loop(0, WPTP, wslab, 0)


@functools.partial(
    pl.kernel,
    out_type=jax.ShapeDtypeStruct((NC, N, D), jnp.float32),
    mesh=_mesh,
    scratch_types=[
        pltpu.VMEM_SHARED((N, D), jnp.float32),
        pltpu.VMEM((NCHUNK, CH), jnp.int32),
        pltpu.VMEM((1, CH), jnp.int32),
        pltpu.VMEM((1, CH), jnp.int32),
        pltpu.VMEM((CH, D), jnp.float32),
        pltpu.VMEM((CH, D), jnp.float32),
        pltpu.SemaphoreType.DMA,
        pltpu.SemaphoreType.DMA,
        pltpu.SemaphoreType.DMA,
        pltpu.SemaphoreType.DMA,
    ],
)
def _sc_prop(g_hbm, rows_hbm, cols_hbm, out_hbm, acc_sh, ridx_v, cidx_a,
             cidx_b, rows_a, rows_b, sem_a, sem_b, sem_ca, sem_cb):
    c = lax.axis_index("c")
    s = lax.axis_index("s")
    w = _worker(c, s)

    zero16 = jnp.zeros((16,), jnp.float32)

    def zrow(i, _):
        def zcol(k, _):
            rows_a[i, pl.ds(k * 16, 16)] = zero16
            return 0
        lax.fori_loop(0, D // 16, zcol, 0)
        return 0

    lax.fori_loop(0, CH, zrow, 0)

    def zslab(k, _):
        cid = k * NS + s

        @pl.when(cid < NWCHP)
        def _():
            pltpu.sync_copy(rows_a.at[pl.ds(0, WCHP)],
                            acc_sh.at[pl.ds(cid * WCHP, WCHP)])
        return 0

    lax.fori_loop(0, WPTP, zslab, 0)
    plsc.subcore_barrier()

    pltpu.sync_copy(rows_hbm.at[w], ridx_v)

    npair = NCHUNK // 2
    cbase = w * NCHUNK
    pltpu.async_copy(g_hbm.at[ridx_v.at[0]], rows_a, sem_a)
    pltpu.async_copy(cols_hbm.at[cbase], cidx_a, sem_ca)
    pltpu.async_copy(cols_hbm.at[cbase + 1], cidx_b, sem_cb)

    def pair(p, _):
        j0 = 2 * p
        j1 = j0 + 1
        pltpu.make_async_copy(g_hbm.at[ridx_v.at[j0]], rows_a, sem_a).wait()
        pltpu.async_copy(g_hbm.at[ridx_v.at[j1]], rows_b, sem_b)
        pltpu.make_async_copy(cols_hbm.at[cbase], cidx_a, sem_ca).wait()
        pltpu.sync_copy(rows_a, acc_sh.at[cidx_a.at[0]], add=True)

        @pl.when(p < npair - 1)
        def _():
            pltpu.async_copy(cols_hbm.at[cbase + j0 + 2], cidx_a, sem_ca)

        pltpu.make_async_copy(g_hbm.at[ridx_v.at[j1]], rows_b, sem_b).wait()

        @pl.when(p < npair - 1)
        def _():
            pltpu.async_copy(g_hbm.at[ridx_v.at[j0 + 2]], rows_a, sem_a)

        pltpu.make_async_copy(cols_hbm.at[cbase], cidx_b, sem_cb).wait()
        pltpu.sync_copy(rows_b, acc_sh.at[cidx_b.at[0]], add=True)

        @pl.when(p < npair - 1)
        def _():
            pltpu.async_copy(cols_hbm.at[cbase + j1 + 2], cidx_b, sem_cb)
        return 0

    lax.fori_loop(0, npair, pair, 0)
    plsc.subcore_barrier()

    def wslab(k, _):
        cid = k * NS + s

        @pl.when(cid < NWCHP)
        def _():
            base = cid * WCHP
            pltpu.sync_copy(acc_sh.at[pl.ds(base, WCHP)],
                            rows_a.at[pl.ds(0, WCHP)])
            pltpu.sync_copy(rows_a.at[pl.ds(0, WCHP)],
                            out_hbm.at[c, pl.ds(base, WCHP)])
        return 0

    lax.fori_loop(0, WPTP, wslab, 0)


BM = 2000
GRID = (N // BM,)


def _tc_pre(degp, x):
    def body(degp_ref, x_ref, dis_ref, g_ref):
        d = degp_ref[0, :, 0:1] + degp_ref[1, :, 0:1]
        dis = jnp.where(d > 0, lax.rsqrt(jnp.maximum(d, 1.0)), 0.0)
        dis_ref[...] = dis
        g_ref[...] = dis * x_ref[...]

    return pl.pallas_call(
        body,
        grid=GRID,
        in_specs=[pl.BlockSpec((NC, BM, D), lambda i: (0, i, 0)),
                  pl.BlockSpec((BM, D), lambda i: (i, 0))],
        out_specs=[pl.BlockSpec((BM, 1), lambda i: (i, 0)),
                   pl.BlockSpec((BM, D), lambda i: (i, 0))],
        out_shape=[jax.ShapeDtypeStruct((N, 1), jnp.float32),
                   jax.ShapeDtypeStruct((N, D), jnp.float32)],
    )(degp, x)


def _tc_comb(sp, dis):
    def body(sp_ref, dis_ref, t_ref, g_ref):
        dis = dis_ref[...]
        t = -dis * (sp_ref[0] + sp_ref[1])
        t_ref[...] = t
        g_ref[...] = dis * t

    return pl.pallas_call(
        body,
        grid=GRID,
        in_specs=[pl.BlockSpec((NC, BM, D), lambda i: (0, i, 0)),
                  pl.BlockSpec((BM, 1), lambda i: (i, 0))],
        out_specs=[pl.BlockSpec((BM, D), lambda i: (i, 0)),
                   pl.BlockSpec((BM, D), lambda i: (i, 0))],
        out_shape=[jax.ShapeDtypeStruct((N, D), jnp.float32),
                   jax.ShapeDtypeStruct((N, D), jnp.float32)],
    )(sp, dis)


def _tc_layer(xin, t1, sp2, dis, w0, w1, w2, b, want_relu):
    def body(x_ref, t1_ref, sp2_ref, dis_ref, w0_ref, w1_ref, w2_ref, b_ref,
             *out_refs):
        x = x_ref[...]
        dis = dis_ref[...]
        t2 = -2.0 * dis * (sp2_ref[0] + sp2_ref[1]) - x
        acc = jnp.dot(x, w0_ref[...], preferred_element_type=jnp.float32)
        acc += jnp.dot(t1_ref[...], w1_ref[...],
                       preferred_element_type=jnp.float32)
        acc += jnp.dot(t2, w2_ref[...], preferred_element_type=jnp.float32)
        acc += b_ref[...]
        if want_relu:
            h = jnp.maximum(acc, 0.0)
            out_refs[0][...] = h
            out_refs[1][...] = dis * h
        else:
            out_refs[0][...] = acc

    nout = 2 if want_relu else 1
    full = pl.BlockSpec((D, D), lambda i: (0, 0))
    res = pl.pallas_call(
        body,
        grid=GRID,
        in_specs=[pl.BlockSpec((BM, D), lambda i: (i, 0)),
                  pl.BlockSpec((BM, D), lambda i: (i, 0)),
                  pl.BlockSpec((NC, BM, D), lambda i: (0, i, 0)),
                  pl.BlockSpec((BM, 1), lambda i: (i, 0)),
                  full, full, full,
                  pl.BlockSpec((1, D), lambda i: (0, 0))],
        out_specs=[pl.BlockSpec((BM, D), lambda i: (i, 0))] * nout,
        out_shape=[jax.ShapeDtypeStruct((N, D), jnp.float32)] * nout,
    )(xin, t1, sp2, dis, w0, w1, w2, b)
    return res


def kernel(x, edge_index, W10, W11, W12, b1, W20, W21, W22, b2):
    rows = edge_index[0].astype(jnp.int32).reshape(NW, NCHUNK, CH)
    cols = edge_index[1].astype(jnp.int32).reshape(NW * NCHUNK, 1, CH)
    b1r = b1.reshape(1, D)
    b2r = b2.reshape(1, D)

    degp = _sc_deg(rows)
    dis, g1 = _tc_pre(degp, x)
    s1 = _sc_prop(g1, rows, cols)
    t1, g2 = _tc_comb(s1, dis)
    s2 = _sc_prop(g2, rows, cols)
    h, g3 = _tc_layer(x, t1, s2, dis, W10, W11, W12, b1r, True)
    s3 = _sc_prop(g3, rows, cols)
    u1, g4 = _tc_comb(s3, dis)
    s4 = _sc_prop(g4, rows, cols)
    (out,) = _tc_layer(h, u1, s4, dis, W20, W21, W22, b2r, False)
    return out
